# Initial kernel scaffold; baseline (speedup 1.0000x reference)
import jax
import jax.numpy as jnp
from jax.experimental import pallas as pl


def kernel(src_feat, tgt_feat, neg_idxs):
    raise NotImplementedError("write your pallas kernel here")


# speedup vs baseline: 4.0485x; 4.0485x over previous
import functools

import jax
import jax.numpy as jnp
from jax import lax
from jax.experimental import pallas as pl
from jax.experimental.pallas import tpu as pltpu
from jax.experimental.pallas import tpu_sc as plsc

B, M, D = 8, 4096, 128
P, K2 = 2048, 10
K = K2 - 2
NC, NS, L = 2, 16, 16
BPC = B // NC
RPT = M // NS
PPT = P // NS
CH = L
NCH = PPT // CH
EPS = 1e-6
MARGIN = 0.5


def _rsqrt16(q):
    i = lax.bitcast_convert_type(q, jnp.int32)
    i = jnp.int32(0x5F3759DF) - (i >> 1)
    y = lax.bitcast_convert_type(i, jnp.float32)
    for _ in range(3):
        y = y * (1.5 - 0.5 * q * y * y)
    return y


_mesh = plsc.VectorSubcoreMesh(core_axis_name="c", subcore_axis_name="s")


@functools.partial(
    pl.kernel,
    out_type=jax.ShapeDtypeStruct((NC, NS, 2, L), jnp.float32),
    mesh=_mesh,
    compiler_params=pltpu.CompilerParams(needs_layout_passes=False),
    scratch_types=[
        pltpu.VMEM_SHARED((M, D), jnp.float32),
        pltpu.VMEM_SHARED((M, D), jnp.float32),
        pltpu.VMEM_SHARED((2, M), jnp.float32),
        pltpu.VMEM((RPT, D), jnp.float32),
        pltpu.VMEM((RPT,), jnp.float32),
        pltpu.VMEM((M,), jnp.float32),
        pltpu.VMEM((M,), jnp.float32),
        pltpu.VMEM((K2, PPT), jnp.int32),
        pltpu.VMEM((CH, D), jnp.float32),
        pltpu.VMEM((CH, D), jnp.float32),
        pltpu.VMEM((K, CH, D), jnp.float32),
        pltpu.VMEM((2, L), jnp.float32),
    ],
)
def _pps_sc(src_hbm, tgt_hbm, idx_hbm, out_hbm,
            src_sp, tgt_sp, scl_sp, rowbuf, sclbuf, scl_src_t, scl_tgt_t,
            idxbuf, abuf, pbuf, nbuf, outbuf):
    c = lax.axis_index("c")
    s = lax.axis_index("s")
    roff = pl.multiple_of(s * RPT, RPT)
    poff = pl.multiple_of(s * PPT, PPT)
    lane = lax.iota(jnp.int32, L)

    def batch_body(bl, accs):
        acc1, acc2 = accs
        b = c * BPC + bl

        for tab, (hbm, sp) in enumerate(((src_hbm, src_sp), (tgt_hbm, tgt_sp))):
            pltpu.sync_copy(hbm.at[b, pl.ds(roff, RPT), :], rowbuf)
            pltpu.sync_copy(rowbuf, sp.at[pl.ds(roff, RPT), :])

            def qgroup(g, _):
                rows = g * L + lane

                def qstep(d, qv):
                    x = plsc.load_gather(
                        rowbuf, [rows, jnp.full((L,), d, jnp.int32)])
                    return qv + x * x

                qvec = lax.fori_loop(
                    0, D, qstep, jnp.zeros((L,), jnp.float32), unroll=4)
                r16 = _rsqrt16(jnp.maximum(qvec, 1e-30))
                goff = pl.multiple_of(g * L, L)
                sclbuf[pl.ds(goff, L)] = jnp.minimum(r16, 1e12)
                return 0

            lax.fori_loop(0, RPT // L, qgroup, 0)
            pltpu.sync_copy(sclbuf, scl_sp.at[tab, pl.ds(roff, RPT)])

        plsc.subcore_barrier()
        pltpu.sync_copy(scl_sp.at[0], scl_src_t)
        pltpu.sync_copy(scl_sp.at[1], scl_tgt_t)
        pltpu.sync_copy(idx_hbm.at[b, :, pl.ds(poff, PPT)], idxbuf)

        def chunk_body(ch, accs):
            acc1, acc2 = accs
            cbase = pl.multiple_of(ch * CH, CH)
            pltpu.sync_copy(src_sp.at[idxbuf.at[0, pl.ds(cbase, CH)]], abuf)
            pltpu.sync_copy(tgt_sp.at[idxbuf.at[1, pl.ds(cbase, CH)]], pbuf)
            for k in range(K):
                pltpu.sync_copy(
                    tgt_sp.at[idxbuf.at[2 + k, pl.ds(cbase, CH)]], nbuf.at[k])
            alpha = plsc.load_gather(scl_src_t, [idxbuf[0, pl.ds(cbase, CH)]])
            pi = plsc.load_gather(scl_tgt_t, [idxbuf[1, pl.ds(cbase, CH)]])
            nus = [
                plsc.load_gather(scl_tgt_t, [idxbuf[2 + k, pl.ds(cbase, CH)]])
                for k in range(K)
            ]

            def dstep(d, acc):
                accp, accn = acc
                dv = jnp.full((L,), d, jnp.int32)
                av = plsc.load_gather(abuf, [lane, dv])
                pv = plsc.load_gather(pbuf, [lane, dv])
                asc = alpha * av + EPS
                t = asc - pi * pv
                accp = accp + t * t
                accn_out = []
                for k in range(K):
                    xk = plsc.load_gather(
                        nbuf, [jnp.full((L,), k, jnp.int32), lane, dv])
                    tk = asc - nus[k] * xk
                    accn_out.append(accn[k] + tk * tk)
                return accp, tuple(accn_out)

            zero = jnp.zeros((L,), jnp.float32)
            accp, accn = lax.fori_loop(
                0, D, dstep, (zero, (zero,) * K), unroll=2)
            dmin = accn[0]
            for k in range(1, K):
                dmin = jnp.minimum(dmin, accn[k])
            acc1 = acc1 + accp
            q = jnp.maximum(dmin, 1e-30)
            dd = q * _rsqrt16(q)
            hin = jnp.maximum(MARGIN - dd, 0.0)
            acc2 = acc2 + hin * hin
            return acc1, acc2

        acc1, acc2 = lax.fori_loop(0, NCH, chunk_body, (acc1, acc2))
        plsc.subcore_barrier()
        return acc1, acc2

    zero = jnp.zeros((L,), jnp.float32)
    acc1, acc2 = lax.fori_loop(0, BPC, batch_body, (zero, zero))
    outbuf[0, :] = acc1
    outbuf[1, :] = acc2
    pltpu.sync_copy(outbuf, out_hbm.at[c, s])


def kernel(src_feat, tgt_feat, neg_idxs):
    idx_t = jnp.transpose(neg_idxs.astype(jnp.int32), (0, 2, 1))
    parts = _pps_sc(src_feat, tgt_feat, idx_t)
    return parts.sum() / jnp.float32(B * P)


# speedup vs baseline: 10.2606x; 2.5344x over previous
import functools

import jax
import jax.numpy as jnp
from jax import lax
from jax.experimental import pallas as pl
from jax.experimental.pallas import tpu as pltpu
from jax.experimental.pallas import tpu_sc as plsc

B, M, D = 8, 4096, 128
P, K2 = 2048, 10
K = K2 - 2
NC, NS, L = 2, 16, 16
BPC = B // NC
RPT = M // NS
PPT = P // NS
CH = L
NCH = PPT // CH
EPS = 1e-6
MARGIN = 0.5


def _rsqrt16(q):
    i = lax.bitcast_convert_type(q, jnp.int32)
    i = jnp.int32(0x5F3759DF) - (i >> 1)
    y = lax.bitcast_convert_type(i, jnp.float32)
    for _ in range(3):
        y = y * (1.5 - 0.5 * q * y * y)
    return y


_mesh = plsc.VectorSubcoreMesh(core_axis_name="c", subcore_axis_name="s")


@functools.partial(
    pl.kernel,
    out_type=jax.ShapeDtypeStruct((NC, NS, 2, L), jnp.float32),
    mesh=_mesh,
    compiler_params=pltpu.CompilerParams(needs_layout_passes=False),
    scratch_types=[
        pltpu.VMEM_SHARED((M, D), jnp.float32),
        pltpu.VMEM_SHARED((M, D), jnp.float32),
        pltpu.VMEM_SHARED((2, M), jnp.float32),
        pltpu.VMEM((RPT, D), jnp.float32),
        pltpu.VMEM((RPT,), jnp.float32),
        pltpu.VMEM((M,), jnp.float32),
        pltpu.VMEM((M,), jnp.float32),
        pltpu.VMEM((K2, PPT), jnp.int32),
        pltpu.VMEM((CH, D), jnp.float32),
        pltpu.VMEM((CH, D), jnp.float32),
        pltpu.VMEM((K, CH, D), jnp.float32),
        pltpu.VMEM((2, L), jnp.float32),
    ],
)
def _pps_sc(src_hbm, tgt_hbm, idx_hbm, out_hbm,
            src_sp, tgt_sp, scl_sp, rowbuf, sclbuf, scl_src_t, scl_tgt_t,
            idxbuf, abuf, pbuf, nbuf, outbuf):
    c = lax.axis_index("c")
    s = lax.axis_index("s")
    roff = pl.multiple_of(s * RPT, RPT)
    poff = pl.multiple_of(s * PPT, PPT)
    lane = lax.iota(jnp.int32, L)

    def batch_body(bl, accs):
        acc1, acc2 = accs
        b = c * BPC + bl

        for tab, (hbm, sp) in enumerate(((src_hbm, src_sp), (tgt_hbm, tgt_sp))):
            pltpu.sync_copy(hbm.at[b, pl.ds(roff, RPT), :], rowbuf)
            pltpu.sync_copy(rowbuf, sp.at[pl.ds(roff, RPT), :])

            def qgroup(g, _):
                rows = g * L + lane

                def qstep(d, qv):
                    dv = (d + 8 * lane) & (D - 1)
                    x = plsc.load_gather(rowbuf, [rows, dv])
                    return qv + x * x

                qvec = lax.fori_loop(
                    0, D, qstep, jnp.zeros((L,), jnp.float32), unroll=4)
                r16 = _rsqrt16(jnp.maximum(qvec, 1e-30))
                goff = pl.multiple_of(g * L, L)
                sclbuf[pl.ds(goff, L)] = jnp.minimum(r16, 1e12)
                return 0

            lax.fori_loop(0, RPT // L, qgroup, 0)
            pltpu.sync_copy(sclbuf, scl_sp.at[tab, pl.ds(roff, RPT)])

        plsc.subcore_barrier()
        pltpu.sync_copy(scl_sp.at[0], scl_src_t)
        pltpu.sync_copy(scl_sp.at[1], scl_tgt_t)
        pltpu.sync_copy(idx_hbm.at[b, :, pl.ds(poff, PPT)], idxbuf)

        def chunk_body(ch, accs):
            acc1, acc2 = accs
            cbase = pl.multiple_of(ch * CH, CH)
            pltpu.sync_copy(src_sp.at[idxbuf.at[0, pl.ds(cbase, CH)]], abuf)
            pltpu.sync_copy(tgt_sp.at[idxbuf.at[1, pl.ds(cbase, CH)]], pbuf)
            for k in range(K):
                pltpu.sync_copy(
                    tgt_sp.at[idxbuf.at[2 + k, pl.ds(cbase, CH)]], nbuf.at[k])
            alpha = plsc.load_gather(scl_src_t, [idxbuf[0, pl.ds(cbase, CH)]])
            pi = plsc.load_gather(scl_tgt_t, [idxbuf[1, pl.ds(cbase, CH)]])
            nus = [
                plsc.load_gather(scl_tgt_t, [idxbuf[2 + k, pl.ds(cbase, CH)]])
                for k in range(K)
            ]

            def dstep(d, acc):
                accp, accn = acc
                dv = (d + 8 * lane) & (D - 1)
                av = plsc.load_gather(abuf, [lane, dv])
                pv = plsc.load_gather(pbuf, [lane, dv])
                asc = alpha * av + EPS
                t = asc - pi * pv
                accp = accp + t * t
                accn_out = []
                for k in range(K):
                    xk = plsc.load_gather(
                        nbuf, [jnp.full((L,), k, jnp.int32), lane, dv])
                    tk = asc - nus[k] * xk
                    accn_out.append(accn[k] + tk * tk)
                return accp, tuple(accn_out)

            zero = jnp.zeros((L,), jnp.float32)
            accp, accn = lax.fori_loop(
                0, D, dstep, (zero, (zero,) * K), unroll=2)
            dmin = accn[0]
            for k in range(1, K):
                dmin = jnp.minimum(dmin, accn[k])
            acc1 = acc1 + accp
            q = jnp.maximum(dmin, 1e-30)
            dd = q * _rsqrt16(q)
            hin = jnp.maximum(MARGIN - dd, 0.0)
            acc2 = acc2 + hin * hin
            return acc1, acc2

        acc1, acc2 = lax.fori_loop(0, NCH, chunk_body, (acc1, acc2))
        plsc.subcore_barrier()
        return acc1, acc2

    zero = jnp.zeros((L,), jnp.float32)
    acc1, acc2 = lax.fori_loop(0, BPC, batch_body, (zero, zero))
    outbuf[0, :] = acc1
    outbuf[1, :] = acc2
    pltpu.sync_copy(outbuf, out_hbm.at[c, s])


def kernel(src_feat, tgt_feat, neg_idxs):
    idx_t = jnp.transpose(neg_idxs.astype(jnp.int32), (0, 2, 1))
    parts = _pps_sc(src_feat, tgt_feat, idx_t)
    return parts.sum() / jnp.float32(B * P)


# speedup vs baseline: 15.5427x; 1.5148x over previous
import functools

import jax
import jax.numpy as jnp
from jax import lax
from jax.experimental import pallas as pl
from jax.experimental.pallas import tpu as pltpu
from jax.experimental.pallas import tpu_sc as plsc

B, M, D = 8, 4096, 128
P, K2 = 2048, 10
K = K2 - 2
NC, NS, L = 2, 16, 16
BPC = B // NC
RPT = M // NS
PPT = P // NS
CH = L
NCH = PPT // CH
EPS = 1e-6
DEPS2 = D * EPS * EPS
MARGIN = 0.5


def _rsqrt16(q):
    i = lax.bitcast_convert_type(q, jnp.int32)
    i = jnp.int32(0x5F3759DF) - (i >> 1)
    y = lax.bitcast_convert_type(i, jnp.float32)
    for _ in range(3):
        y = y * (1.5 - 0.5 * q * y * y)
    return y


_mesh = plsc.VectorSubcoreMesh(core_axis_name="c", subcore_axis_name="s")


@functools.partial(
    pl.kernel,
    out_type=jax.ShapeDtypeStruct((NC, NS, 2, L), jnp.float32),
    mesh=_mesh,
    compiler_params=pltpu.CompilerParams(needs_layout_passes=False),
    scratch_types=[
        pltpu.VMEM_SHARED((4, M), jnp.float32),
        pltpu.VMEM((RPT, D), jnp.float32),
        pltpu.VMEM((2, RPT), jnp.float32),
        pltpu.VMEM((M,), jnp.float32),
        pltpu.VMEM((M,), jnp.float32),
        pltpu.VMEM((M,), jnp.float32),
        pltpu.VMEM((M,), jnp.float32),
        pltpu.VMEM((K2, PPT), jnp.int32),
        pltpu.VMEM((2, K2, CH, D), jnp.float32),
        pltpu.VMEM((2, L), jnp.float32),
        pltpu.SemaphoreType.DMA,
        pltpu.SemaphoreType.DMA,
    ],
)
def _pps_sc(src_hbm, tgt_hbm, idx_hbm, out_hbm,
            scl_sp, rowbuf, cbuf,
            t_ss, t_su, t_ts, t_tv, idxbuf, rbuf, outbuf,
            sem0, sem1):
    c = lax.axis_index("c")
    s = lax.axis_index("s")
    roff = pl.multiple_of(s * RPT, RPT)
    poff = pl.multiple_of(s * PPT, PPT)
    lane = lax.iota(jnp.int32, L)
    sems = (sem0, sem1)

    def batch_body(bl, accs):
        acc1, acc2 = accs
        b = c * BPC + bl

        def issue(ch):
            slot = ch & 1
            cb = ch * CH
            hs = [pltpu.async_copy(
                src_hbm.at[b].at[idxbuf.at[0, pl.ds(cb, CH)]],
                rbuf.at[slot, 0], sems[slot])]
            for j in range(1, K2):
                hs.append(pltpu.async_copy(
                    tgt_hbm.at[b].at[idxbuf.at[j, pl.ds(cb, CH)]],
                    rbuf.at[slot, j], sems[slot]))
            return hs

        pltpu.sync_copy(idx_hbm.at[b, :, pl.ds(poff, PPT)], idxbuf)
        hs = issue(0)

        for tab, hbm in enumerate((src_hbm, tgt_hbm)):
            pltpu.sync_copy(hbm.at[b, pl.ds(roff, RPT), :], rowbuf)

            def qgroup(g, _):
                rows = g * L + lane

                def qstep(d, acc):
                    qv, sv = acc
                    dv = (d + 8 * lane) & (D - 1)
                    x = plsc.load_gather(rowbuf, [rows, dv])
                    return qv + x * x, sv + x

                zero = jnp.zeros((L,), jnp.float32)
                qv, sv = lax.fori_loop(0, D, qstep, (zero, zero), unroll=4)
                scale = jnp.minimum(
                    _rsqrt16(jnp.maximum(qv, 1e-30)), 1e12)
                sq = scale * scale * qv
                es = (2.0 * EPS) * scale * sv
                uv = sq + es if tab == 0 else sq - es
                goff = pl.multiple_of(g * L, L)
                cbuf[0, pl.ds(goff, L)] = scale
                cbuf[1, pl.ds(goff, L)] = uv
                return 0

            lax.fori_loop(0, RPT // L, qgroup, 0)
            pltpu.sync_copy(
                cbuf, scl_sp.at[pl.ds(2 * tab, 2), pl.ds(roff, RPT)])

        plsc.subcore_barrier()
        pltpu.sync_copy(scl_sp.at[0], t_ss)
        pltpu.sync_copy(scl_sp.at[1], t_su)
        pltpu.sync_copy(scl_sp.at[2], t_ts)
        pltpu.sync_copy(scl_sp.at[3], t_tv)

        def compute(ch):
            slot = ch & 1
            cb = ch * CH
            i0 = idxbuf[0, pl.ds(cb, CH)]
            i1 = idxbuf[1, pl.ds(cb, CH)]
            sa = plsc.load_gather(t_ss, [i0])
            ua = plsc.load_gather(t_su, [i0])
            sp_ = plsc.load_gather(t_ts, [i1])
            vp = plsc.load_gather(t_tv, [i1])
            sns, vns = [], []
            for k in range(K):
                ik = idxbuf[2 + k, pl.ds(cb, CH)]
                sns.append(plsc.load_gather(t_ts, [ik]))
                vns.append(plsc.load_gather(t_tv, [ik]))
            cs = jnp.full((L,), slot, jnp.int32)
            cj = [jnp.full((L,), j, jnp.int32) for j in range(K2)]

            def dstep(d, acc):
                dp, dn = acc
                dv = (d + 8 * lane) & (D - 1)
                av = plsc.load_gather(rbuf, [cs, cj[0], lane, dv])
                dp = dp + av * plsc.load_gather(rbuf, [cs, cj[1], lane, dv])
                dn_out = []
                for k in range(K):
                    xk = plsc.load_gather(rbuf, [cs, cj[2 + k], lane, dv])
                    dn_out.append(dn[k] + av * xk)
                return dp, tuple(dn_out)

            zero = jnp.zeros((L,), jnp.float32)
            dp, dn = lax.fori_loop(0, D, dstep, (zero, (zero,) * K), unroll=2)
            base = ua + DEPS2
            sa2 = sa + sa
            d2p = base + vp - (sa2 * sp_) * dp
            dmin = base + vns[0] - (sa2 * sns[0]) * dn[0]
            for k in range(1, K):
                dmin = jnp.minimum(dmin, base + vns[k] - (sa2 * sns[k]) * dn[k])
            q = jnp.maximum(dmin, 1e-30)
            dd = q * _rsqrt16(q)
            hin = jnp.maximum(MARGIN - dd, 0.0)
            return d2p, hin * hin

        for ch in range(NCH):
            nxt = issue(ch + 1) if ch + 1 < NCH else None
            for h in hs:
                h.wait()
            d1, d2 = compute(ch)
            acc1 = acc1 + d1
            acc2 = acc2 + d2
            hs = nxt

        plsc.subcore_barrier()
        return acc1, acc2

    zero = jnp.zeros((L,), jnp.float32)
    acc1, acc2 = lax.fori_loop(0, BPC, batch_body, (zero, zero))
    outbuf[0, :] = acc1
    outbuf[1, :] = acc2
    pltpu.sync_copy(outbuf, out_hbm.at[c, s])


def kernel(src_feat, tgt_feat, neg_idxs):
    idx_t = jnp.transpose(neg_idxs.astype(jnp.int32), (0, 2, 1))
    parts = _pps_sc(src_feat, tgt_feat, idx_t)
    return parts.sum() / jnp.float32(B * P)
